# Initial kernel scaffold; baseline (speedup 1.0000x reference)
import jax
import jax.numpy as jnp
from jax.experimental import pallas as pl


def kernel(hidden_states, ln1_w, ln2_w, w_qkv, w_o, router_w, w1, w2):
    raise NotImplementedError("write your pallas kernel here")


# speedup vs baseline: 1.8431x; 1.8431x over previous
import functools

import jax
import jax.numpy as jnp
from jax.experimental import pallas as pl
from jax.experimental.pallas import tpu as pltpu

S, B, H = 2048, 1, 768
NH, DH = 12, 64
E, K, F = 8, 2, 1024
EPS = 1e-06
BT = 256


def _qkv_rope_kernel(hs_ref, ln1_ref, wqkv_ref, cos_ref, sin_ref,
                     q_ref, k_ref, v_ref):
    x = hs_ref[...]
    var = jnp.mean(x * x, axis=-1, keepdims=True)
    xn = x * jax.lax.rsqrt(var + EPS) * ln1_ref[...]
    qkv = jnp.dot(xn, wqkv_ref[...], preferred_element_type=jnp.float32)
    cos = cos_ref[...]
    sin = sin_ref[...]
    q_parts, k_parts, v_parts = [], [], []
    for h in range(NH):
        base = h * 3 * DH
        qh = qkv[:, base:base + DH]
        kh = qkv[:, base + DH:base + 2 * DH]
        vh = qkv[:, base + 2 * DH:base + 3 * DH]
        half = DH // 2
        qrot = jnp.concatenate([-qh[:, half:], qh[:, :half]], axis=1)
        krot = jnp.concatenate([-kh[:, half:], kh[:, :half]], axis=1)
        q_parts.append(qh * cos + qrot * sin)
        k_parts.append(kh * cos + krot * sin)
        v_parts.append(vh)
    q_ref[...] = jnp.concatenate(q_parts, axis=1)
    k_ref[...] = jnp.concatenate(k_parts, axis=1)
    v_ref[...] = jnp.concatenate(v_parts, axis=1)


def _attn_kernel(q_ref, k_ref, v_ref, ctx_ref):
    i = pl.program_id(0)
    row = jax.lax.broadcasted_iota(jnp.int32, (BT, S), 0) + i * BT
    col = jax.lax.broadcasted_iota(jnp.int32, (BT, S), 1)
    mask = col <= row
    scale = 1.0 / (DH ** 0.5)
    parts = []
    for h in range(NH):
        qh = q_ref[:, h * DH:(h + 1) * DH]
        kh = k_ref[:, h * DH:(h + 1) * DH]
        s = jax.lax.dot_general(qh, kh, (((1,), (1,)), ((), ())),
                                preferred_element_type=jnp.float32) * scale
        s = jnp.where(mask, s, -1e9)
        m = jnp.max(s, axis=-1, keepdims=True)
        p = jnp.exp(s - m)
        p = p / jnp.sum(p, axis=-1, keepdims=True)
        parts.append(jnp.dot(p, v_ref[:, h * DH:(h + 1) * DH],
                             preferred_element_type=jnp.float32))
    ctx_ref[...] = jnp.concatenate(parts, axis=1)


def _proj_router_kernel(ctx_ref, resid_ref, wo_ref, ln2_ref, wr_ref,
                        attn_out_ref, h2_ref, probs_ref):
    attn_out = jnp.dot(ctx_ref[...], wo_ref[...],
                       preferred_element_type=jnp.float32) + resid_ref[...]
    attn_out_ref[...] = attn_out
    var = jnp.mean(attn_out * attn_out, axis=-1, keepdims=True)
    h2 = attn_out * jax.lax.rsqrt(var + EPS) * ln2_ref[...]
    h2_ref[...] = h2
    logits = jnp.dot(h2, wr_ref[...], preferred_element_type=jnp.float32)
    eio = jax.lax.broadcasted_iota(jnp.int32, (BT, E), 1)
    m1 = jnp.max(logits, axis=-1, keepdims=True)
    i1 = jnp.min(jnp.where(logits == m1, eio, E), axis=-1, keepdims=True)
    l2 = jnp.where(eio == i1, -jnp.inf, logits)
    m2 = jnp.max(l2, axis=-1, keepdims=True)
    i2 = jnp.min(jnp.where(l2 == m2, eio, E), axis=-1, keepdims=True)
    z = jnp.exp(m2 - m1)
    p1 = 1.0 / (1.0 + z)
    p2 = 1.0 - p1
    probs_ref[...] = (jnp.where(eio == i1, p1, 0.0)
                      + jnp.where(eio == i2, p2, 0.0))


def _moe_kernel(h2_ref, res_ref, probs_ref, w1_ref, w2_ref, out_ref):
    e = pl.program_id(0)

    @pl.when(e == 0)
    def _():
        out_ref[...] = res_ref[...]

    x = h2_ref[...]
    a = jnp.dot(x, w1_ref[0], preferred_element_type=jnp.float32)
    g = jax.nn.gelu(a)
    y = jnp.dot(g, w2_ref[0], preferred_element_type=jnp.float32)
    eio = jax.lax.broadcasted_iota(jnp.int32, (S, E), 1)
    w = jnp.sum(jnp.where(eio == e, probs_ref[...], 0.0),
                axis=-1, keepdims=True)
    out_ref[...] = out_ref[...] + w * y


def kernel(hidden_states, ln1_w, ln2_w, w_qkv, w_o, router_w, w1, w2):
    hs = hidden_states.reshape(S, H)
    ln1 = ln1_w.reshape(1, H)
    ln2 = ln2_w.reshape(1, H)

    inv_freq = 1.0 / (10000.0 ** (jnp.arange(0, DH, 2, dtype=jnp.float32) / DH))
    t = jnp.arange(S, dtype=jnp.float32)
    freqs = jnp.outer(t, inv_freq)
    emb = jnp.concatenate([freqs, freqs], axis=-1)
    cos, sin = jnp.cos(emb), jnp.sin(emb)

    nT = S // BT
    f32 = jnp.float32

    q, k, v = pl.pallas_call(
        _qkv_rope_kernel,
        grid=(nT,),
        in_specs=[
            pl.BlockSpec((BT, H), lambda i: (i, 0)),
            pl.BlockSpec((1, H), lambda i: (0, 0)),
            pl.BlockSpec((H, 3 * H), lambda i: (0, 0)),
            pl.BlockSpec((BT, DH), lambda i: (i, 0)),
            pl.BlockSpec((BT, DH), lambda i: (i, 0)),
        ],
        out_specs=[pl.BlockSpec((BT, H), lambda i: (i, 0))] * 3,
        out_shape=[jax.ShapeDtypeStruct((S, H), f32)] * 3,
    )(hs, ln1, w_qkv, cos, sin)

    ctx = pl.pallas_call(
        _attn_kernel,
        grid=(nT,),
        in_specs=[
            pl.BlockSpec((BT, H), lambda i: (i, 0)),
            pl.BlockSpec((S, H), lambda i: (0, 0)),
            pl.BlockSpec((S, H), lambda i: (0, 0)),
        ],
        out_specs=pl.BlockSpec((BT, H), lambda i: (i, 0)),
        out_shape=jax.ShapeDtypeStruct((S, H), f32),
    )(q, k, v)

    attn_out, h2, probs = pl.pallas_call(
        _proj_router_kernel,
        grid=(nT,),
        in_specs=[
            pl.BlockSpec((BT, H), lambda i: (i, 0)),
            pl.BlockSpec((BT, H), lambda i: (i, 0)),
            pl.BlockSpec((H, H), lambda i: (0, 0)),
            pl.BlockSpec((1, H), lambda i: (0, 0)),
            pl.BlockSpec((H, E), lambda i: (0, 0)),
        ],
        out_specs=[
            pl.BlockSpec((BT, H), lambda i: (i, 0)),
            pl.BlockSpec((BT, H), lambda i: (i, 0)),
            pl.BlockSpec((BT, E), lambda i: (i, 0)),
        ],
        out_shape=[
            jax.ShapeDtypeStruct((S, H), f32),
            jax.ShapeDtypeStruct((S, H), f32),
            jax.ShapeDtypeStruct((S, E), f32),
        ],
    )(ctx, hs, w_o, ln2, router_w)

    out = pl.pallas_call(
        _moe_kernel,
        grid=(E,),
        in_specs=[
            pl.BlockSpec((S, H), lambda e: (0, 0)),
            pl.BlockSpec((S, H), lambda e: (0, 0)),
            pl.BlockSpec((S, E), lambda e: (0, 0)),
            pl.BlockSpec((1, H, F), lambda e: (e, 0, 0)),
            pl.BlockSpec((1, F, H), lambda e: (e, 0, 0)),
        ],
        out_specs=pl.BlockSpec((S, H), lambda e: (0, 0)),
        out_shape=jax.ShapeDtypeStruct((S, H), f32),
    )(h2, attn_out, probs, w1, w2)

    return out.reshape(S, B, H)
